# Initial kernel scaffold; baseline (speedup 1.0000x reference)
import jax
import jax.numpy as jnp
from jax.experimental import pallas as pl


def kernel(inp_word, inp_char, table):
    raise NotImplementedError("write your pallas kernel here")


# speedup vs baseline: 2.8965x; 2.8965x over previous
import functools

import jax
import jax.numpy as jnp
from jax import lax
from jax.experimental import pallas as pl
from jax.experimental.pallas import tpu as pltpu
from jax.experimental.pallas import tpu_sc as plsc

VOCAB = 100000
EMB = 128
B = 4096
L = 50

_INFO = plsc.get_sparse_core_info()
NC = _INFO.num_cores
NS = _INFO.num_subcores
NW = NC * NS
TOTAL = B * L
BPW = TOTAL // NW
CHUNK = 128
NCH = BPW // CHUNK

_mesh = plsc.VectorSubcoreMesh(core_axis_name="c", subcore_axis_name="s")


@functools.partial(
    pl.kernel,
    mesh=_mesh,
    out_type=jax.ShapeDtypeStruct((TOTAL, EMB), jnp.float32),
    scratch_types=[
        pltpu.VMEM((NCH, CHUNK), jnp.int32),
        pltpu.VMEM((CHUNK, EMB), jnp.float32),
        pltpu.SemaphoreType.DMA,
    ],
)
def _gather_kernel(idx_hbm, table_hbm, out_hbm, idx_v, rows_v, sem):
    wid = lax.axis_index("s") * NC + lax.axis_index("c")
    pltpu.sync_copy(idx_hbm.at[wid], idx_v)
    base = wid * BPW

    def body(j, carry):
        pltpu.async_copy(table_hbm.at[idx_v.at[j]], rows_v, sem).wait()
        pltpu.sync_copy(rows_v, out_hbm.at[pl.ds(base + j * CHUNK, CHUNK)])
        return carry

    lax.fori_loop(0, NCH, body, 0)


def kernel(inp_word, inp_char, table):
    idx = inp_word.reshape(NW, NCH, CHUNK)
    x = _gather_kernel(idx, table)
    return (x.reshape(B, L, EMB), inp_word, inp_char)


# speedup vs baseline: 3.2259x; 1.1137x over previous
import functools

import jax
import jax.numpy as jnp
from jax import lax
from jax.experimental import pallas as pl
from jax.experimental.pallas import tpu as pltpu
from jax.experimental.pallas import tpu_sc as plsc

VOCAB = 100000
EMB = 128
B = 4096
L = 50

_INFO = plsc.get_sparse_core_info()
NC = _INFO.num_cores
NS = _INFO.num_subcores
NW = NC * NS
TOTAL = B * L
BPW = TOTAL // NW
CHUNK = 128
NCH = BPW // CHUNK
NBUF = 4

_mesh = plsc.VectorSubcoreMesh(core_axis_name="c", subcore_axis_name="s")


@functools.partial(
    pl.kernel,
    mesh=_mesh,
    out_type=jax.ShapeDtypeStruct((TOTAL, EMB), jnp.float32),
    scratch_types=[
        pltpu.VMEM((NCH, CHUNK), jnp.int32),
        pltpu.VMEM((NBUF, CHUNK, EMB), jnp.float32),
    ]
    + [pltpu.SemaphoreType.DMA] * (2 * NBUF),
)
def _gather_kernel(idx_hbm, table_hbm, out_hbm, idx_v, rows_v, *sems):
    gsem = sems[:NBUF]
    ssem = sems[NBUF:]
    wid = lax.axis_index("s") * NC + lax.axis_index("c")
    pltpu.sync_copy(idx_hbm.at[wid], idx_v)
    base = wid * BPW

    def fire_gather(g, b):
        pltpu.async_copy(table_hbm.at[idx_v.at[g]], rows_v.at[b], gsem[b])

    def wait_gather(g, b):
        pltpu.make_async_copy(
            table_hbm.at[idx_v.at[g]], rows_v.at[b], gsem[b]
        ).wait()

    def out_slice(g):
        return out_hbm.at[pl.ds(base + g * CHUNK, CHUNK)]

    def fire_store(g, b):
        pltpu.async_copy(rows_v.at[b], out_slice(g), ssem[b])

    def wait_store(g, b):
        pltpu.make_async_copy(rows_v.at[b], out_slice(g), ssem[b]).wait()

    def pre(g, b, with_wait):
        if with_wait:
            wait_store(g - NBUF, b)
        fire_gather(g, b)

    def post(g, b):
        wait_gather(g, b)
        fire_store(g, b)

    fire_gather(0, 0)
    fire_gather(1, 1)
    for b in range(NBUF):
        g = b
        pre(g + 2, (g + 2) % NBUF, with_wait=(g + 2) >= NBUF)
        post(g, b)

    def body(q, carry):
        for b in range(NBUF):
            g = q * NBUF + b
            pre(g + 2, (b + 2) % NBUF, with_wait=True)
            post(g, b)
        return carry

    lax.fori_loop(1, (NCH - 2) // NBUF, body, 0)

    post(NCH - 2, (NCH - 2) % NBUF)
    post(NCH - 1, (NCH - 1) % NBUF)
    for g in range(NCH - NBUF, NCH):
        wait_store(g, g % NBUF)


def kernel(inp_word, inp_char, table):
    idx = inp_word.reshape(NW, NCH, CHUNK)
    x = _gather_kernel(idx, table)
    return (x.reshape(B, L, EMB), inp_word, inp_char)


# speedup vs baseline: 5.4999x; 1.7049x over previous
import functools

import jax
import jax.numpy as jnp
from jax import lax
from jax.experimental import pallas as pl
from jax.experimental.pallas import tpu as pltpu
from jax.experimental.pallas import tpu_sc as plsc

VOCAB = 100000
EMB = 128
B = 4096
L = 50

_INFO = plsc.get_sparse_core_info()
NC = _INFO.num_cores
NS = _INFO.num_subcores
NW = NC * NS
SPW = B // NW
NCH = SPW
NBUF = 4
LEAD = 2
EPI = (NCH - LEAD) % NBUF + LEAD

_mesh = plsc.VectorSubcoreMesh(core_axis_name="c", subcore_axis_name="s")


@functools.partial(
    pl.kernel,
    mesh=_mesh,
    out_type=jax.ShapeDtypeStruct((B, L, EMB), jnp.float32),
    scratch_types=[
        pltpu.VMEM((SPW, L), jnp.int32),
        pltpu.VMEM((NBUF, L, EMB), jnp.float32),
    ]
    + [pltpu.SemaphoreType.DMA] * (2 * NBUF),
)
def _gather_kernel(idx_hbm, table_hbm, out_hbm, idx_v, rows_v, *sems):
    gsem = sems[:NBUF]
    ssem = sems[NBUF:]
    wid = lax.axis_index("s") * NC + lax.axis_index("c")
    pltpu.sync_copy(idx_hbm.at[wid], idx_v)
    base = wid * SPW

    def fire_gather(g, b):
        pltpu.async_copy(table_hbm.at[idx_v.at[g]], rows_v.at[b], gsem[b])

    def wait_gather(g, b):
        pltpu.make_async_copy(
            table_hbm.at[idx_v.at[g]], rows_v.at[b], gsem[b]
        ).wait()

    def fire_store(g, b):
        pltpu.async_copy(rows_v.at[b], out_hbm.at[base + g], ssem[b])

    def wait_store(g, b):
        pltpu.make_async_copy(rows_v.at[b], out_hbm.at[base + g], ssem[b]).wait()

    def pre(g, b, with_wait):
        if with_wait:
            wait_store(g - NBUF, b)
        fire_gather(g, b)

    def post(g, b):
        wait_gather(g, b)
        fire_store(g, b)

    for g in range(LEAD):
        fire_gather(g, g % NBUF)
    for b in range(NBUF):
        g = b
        pre(g + LEAD, (g + LEAD) % NBUF, with_wait=(g + LEAD) >= NBUF)
        post(g, b)

    def body(q, carry):
        for b in range(NBUF):
            g = q * NBUF + b
            pre(g + LEAD, (b + LEAD) % NBUF, with_wait=True)
            post(g, b)
        return carry

    lax.fori_loop(1, (NCH - EPI) // NBUF, body, 0)

    for g in range(NCH - EPI, NCH):
        if g + LEAD < NCH:
            pre(g + LEAD, (g + LEAD) % NBUF, with_wait=True)
        post(g, g % NBUF)
    for g in range(NCH - NBUF, NCH):
        wait_store(g, g % NBUF)


def kernel(inp_word, inp_char, table):
    idx = inp_word.reshape(NW, SPW, L)
    x = _gather_kernel(idx, table)
    return (x, inp_word, inp_char)


# speedup vs baseline: 9.3143x; 1.6936x over previous
import functools

import jax
import jax.numpy as jnp
from jax import lax
from jax.experimental import pallas as pl
from jax.experimental.pallas import tpu as pltpu
from jax.experimental.pallas import tpu_sc as plsc

VOCAB = 100000
EMB = 128
B = 4096
L = 50

_INFO = plsc.get_sparse_core_info()
NC = _INFO.num_cores
NS = _INFO.num_subcores
NW = NC * NS
TOTAL = B * L
BPW = TOTAL // NW
CHUNK = 128
NCH = BPW // CHUNK
NBUF = 6
LEAD = 4
EPI = (NCH - LEAD) % NBUF + LEAD

_mesh = plsc.VectorSubcoreMesh(core_axis_name="c", subcore_axis_name="s")


@functools.partial(
    pl.kernel,
    mesh=_mesh,
    out_type=jax.ShapeDtypeStruct((TOTAL, EMB), jnp.float32),
    scratch_types=[
        pltpu.VMEM((NCH, CHUNK), jnp.int32),
        pltpu.VMEM((NBUF, CHUNK, EMB), jnp.float32),
    ]
    + [pltpu.SemaphoreType.DMA] * (2 * NBUF),
)
def _gather_kernel(idx_hbm, table_hbm, out_hbm, idx_v, rows_v, *sems):
    gsem = sems[:NBUF]
    ssem = sems[NBUF:]
    wid = lax.axis_index("s") * NC + lax.axis_index("c")
    pltpu.sync_copy(idx_hbm.at[wid], idx_v)
    base = wid * BPW

    def fire_gather(g, b):
        pltpu.async_copy(table_hbm.at[idx_v.at[g]], rows_v.at[b], gsem[b])

    def wait_gather(g, b):
        pltpu.make_async_copy(
            table_hbm.at[idx_v.at[g]], rows_v.at[b], gsem[b]
        ).wait()

    def fire_store(g, b):
        pltpu.async_copy(
            rows_v.at[b], out_hbm.at[pl.ds(base + g * CHUNK, CHUNK)], ssem[b]
        )

    def wait_store(g, b):
        pltpu.make_async_copy(
            rows_v.at[b], out_hbm.at[pl.ds(base + g * CHUNK, CHUNK)], ssem[b]
        ).wait()

    def pre(g, b, with_wait):
        if with_wait:
            wait_store(g - NBUF, b)
        fire_gather(g, b)

    def post(g, b):
        wait_gather(g, b)
        fire_store(g, b)

    for g in range(LEAD):
        fire_gather(g, g % NBUF)
    for b in range(NBUF):
        g = b
        pre(g + LEAD, (g + LEAD) % NBUF, with_wait=(g + LEAD) >= NBUF)
        post(g, b)

    def body(q, carry):
        for b in range(NBUF):
            g = q * NBUF + b
            pre(g + LEAD, (b + LEAD) % NBUF, with_wait=True)
            post(g, b)
        return carry

    lax.fori_loop(1, (NCH - EPI) // NBUF, body, 0)

    for g in range(NCH - EPI, NCH):
        if g + LEAD < NCH:
            pre(g + LEAD, (g + LEAD) % NBUF, with_wait=True)
        post(g, g % NBUF)
    for g in range(NCH - NBUF, NCH):
        wait_store(g, g % NBUF)


def kernel(inp_word, inp_char, table):
    idx = inp_word.T.reshape(NW, NCH, CHUNK)
    x = _gather_kernel(idx, table)
    x = x.reshape(L, B, EMB).transpose(1, 0, 2)
    return (x, inp_word, inp_char)
